# baseline (speedup 1.0000x reference)
import functools

import jax
import jax.numpy as jnp
from jax import lax
from jax.experimental import pallas as pl
from jax.experimental.pallas import tpu as pltpu
from jax.experimental.pallas import tpu_sc as plsc

_B = 16384
_D = 64
_L = 16
_NC = 2
_NS = 16
_NW = _NC * _NS
_BPW = _B // _NW
_CHUNK = 128
_NCHUNK = _BPW // _CHUNK


@functools.partial(
    pl.kernel,
    mesh=plsc.VectorSubcoreMesh(core_axis_name="c", subcore_axis_name="s"),
    out_type=jax.ShapeDtypeStruct((_B,), jnp.float32),
    compiler_params=pltpu.CompilerParams(use_tc_tiling_on_sc=False),
    scratch_types=[
        pltpu.VMEM((_NCHUNK, _CHUNK), jnp.int32),
        pltpu.VMEM((_NCHUNK, _CHUNK), jnp.int32),
        pltpu.VMEM((_BPW, _D), jnp.float32),
        pltpu.VMEM((_BPW, _D), jnp.float32),
        pltpu.VMEM((_D,), jnp.float32),
        pltpu.VMEM((_L,), jnp.float32),
        pltpu.VMEM((_BPW,), jnp.float32),
        pltpu.SemaphoreType.DMA,
    ],
)
def _gmf_sc(user_hbm, item_hbm, iu_hbm, iv_hbm, w_hbm, bias_hbm, out_hbm,
            iu_v, iv_v, u_rows, v_rows, w_v, bias_v, out_v, sem):
    wid = lax.axis_index("s") * _NC + lax.axis_index("c")
    base = wid * _BPW

    pltpu.sync_copy(iu_hbm.at[pl.ds(wid * _NCHUNK, _NCHUNK)], iu_v)
    pltpu.sync_copy(iv_hbm.at[pl.ds(wid * _NCHUNK, _NCHUNK)], iv_v)
    pltpu.sync_copy(w_hbm, w_v)
    pltpu.sync_copy(bias_hbm, bias_v)

    copies = []
    for j in range(_NCHUNK):
        copies.append(pltpu.async_copy(
            user_hbm.at[iu_v.at[j]], u_rows.at[pl.ds(j * _CHUNK, _CHUNK)], sem))
        copies.append(pltpu.async_copy(
            item_hbm.at[iv_v.at[j]], v_rows.at[pl.ds(j * _CHUNK, _CHUNK)], sem))
    for c in copies:
        c.wait()

    w0 = w_v[pl.ds(0 * _L, _L)]
    w1 = w_v[pl.ds(1 * _L, _L)]
    w2 = w_v[pl.ds(2 * _L, _L)]
    w3 = w_v[pl.ds(3 * _L, _L)]

    lane = lax.iota(jnp.int32, _L)
    lo_half = lane < (_L // 2)
    perm_even = (lane * 2) % _L
    perm_odd = perm_even + 1

    def shuf(x, perm):
        return lax.gather(
            x, perm[:, None],
            lax.GatherDimensionNumbers(
                offset_dims=(), collapsed_slice_dims=(0,), start_index_map=(0,)),
            slice_sizes=(1,),
            mode=lax.GatherScatterMode.PROMISE_IN_BOUNDS)

    def hadd(a, b):
        return jnp.where(lo_half,
                         shuf(a, perm_even) + shuf(a, perm_odd),
                         shuf(b, perm_even) + shuf(b, perm_odd))

    def block_body(blk, carry):
        base_r = blk * _L
        ps = []
        for k in range(_L):
            r = base_r + k
            p = (u_rows[r, pl.ds(0 * _L, _L)] * w0) * v_rows[r, pl.ds(0 * _L, _L)]
            p = p + (u_rows[r, pl.ds(1 * _L, _L)] * w1) * v_rows[r, pl.ds(1 * _L, _L)]
            p = p + (u_rows[r, pl.ds(2 * _L, _L)] * w2) * v_rows[r, pl.ds(2 * _L, _L)]
            p = p + (u_rows[r, pl.ds(3 * _L, _L)] * w3) * v_rows[r, pl.ds(3 * _L, _L)]
            ps.append(p)
        while len(ps) > 1:
            ps = [hadd(ps[i], ps[i + 1]) for i in range(0, len(ps), 2)]
        out_v[pl.ds(base_r, _L)] = ps[0]
        return carry

    lax.fori_loop(0, _BPW // _L, block_body, 0)

    bv = bias_v[...]
    for i in range(_BPW // _L):
        x = out_v[pl.ds(i * _L, _L)] + bv
        out_v[pl.ds(i * _L, _L)] = 1.0 / (1.0 + jnp.exp(-x))

    pltpu.sync_copy(out_v, out_hbm.at[pl.ds(base, _BPW)])


def kernel(inputs, user_table, item_table, W, b):
    idx = inputs.astype(jnp.int32)
    iu = idx[:, 0].reshape(_NW * _NCHUNK, _CHUNK)
    iv = idx[:, 1].reshape(_NW * _NCHUNK, _CHUNK)
    w64 = W.reshape(_D).astype(jnp.float32)
    bias = jnp.broadcast_to(b.astype(jnp.float32), (_L,))
    out = _gmf_sc(user_table, item_table, iu, iv, w64, bias)
    return out.reshape(_B, 1)


# speedup vs baseline: 1.0257x; 1.0257x over previous
import functools

import jax
import jax.numpy as jnp
from jax import lax
from jax.experimental import pallas as pl
from jax.experimental.pallas import tpu as pltpu
from jax.experimental.pallas import tpu_sc as plsc

_B = 16384
_D = 64
_L = 16
_NC = 2
_NS = 16
_NW = _NC * _NS
_BPW = _B // _NW
_DP = 128
_CHUNK = 128
_NCHUNK = _BPW // _CHUNK


@functools.partial(
    pl.kernel,
    mesh=plsc.VectorSubcoreMesh(core_axis_name="c", subcore_axis_name="s"),
    out_type=jax.ShapeDtypeStruct((_B,), jnp.float32),
    compiler_params=pltpu.CompilerParams(use_tc_tiling_on_sc=True),
    scratch_types=[
        pltpu.VMEM((_NCHUNK, _CHUNK), jnp.int32),
        pltpu.VMEM((_NCHUNK, _CHUNK), jnp.int32),
        pltpu.VMEM((_CHUNK, _DP), jnp.float32),
        pltpu.VMEM((_CHUNK, _DP), jnp.float32),
        pltpu.VMEM((_CHUNK, _DP), jnp.float32),
        pltpu.VMEM((_CHUNK, _DP), jnp.float32),
        pltpu.VMEM((_D,), jnp.float32),
        pltpu.VMEM((_L,), jnp.float32),
        pltpu.VMEM((_BPW,), jnp.float32),
        pltpu.SemaphoreType.DMA,
        pltpu.SemaphoreType.DMA,
    ],
)
def _gmf_sc(user_hbm, item_hbm, iu_hbm, iv_hbm, w_hbm, bias_hbm, out_hbm,
            iu_v, iv_v, ub0, ub1, vb0, vb1, w_v, bias_v, out_v, sem0, sem1):
    wid = lax.axis_index("s") * _NC + lax.axis_index("c")
    base = wid * _BPW
    ubufs = (ub0, ub1)
    vbufs = (vb0, vb1)
    sems = (sem0, sem1)

    pltpu.sync_copy(iu_hbm.at[pl.ds(wid * _NCHUNK, _NCHUNK)], iu_v)
    pltpu.sync_copy(iv_hbm.at[pl.ds(wid * _NCHUNK, _NCHUNK)], iv_v)
    pltpu.sync_copy(w_hbm, w_v)
    pltpu.sync_copy(bias_hbm, bias_v)

    def fire(j):
        k = j % 2
        return (pltpu.async_copy(user_hbm.at[iu_v.at[j]], ubufs[k], sems[k]),
                pltpu.async_copy(item_hbm.at[iv_v.at[j]], vbufs[k], sems[k]))

    w0 = w_v[pl.ds(0 * _L, _L)]
    w1 = w_v[pl.ds(1 * _L, _L)]
    w2 = w_v[pl.ds(2 * _L, _L)]
    w3 = w_v[pl.ds(3 * _L, _L)]

    lane = lax.iota(jnp.int32, _L)
    lo_half = lane < (_L // 2)
    perm_even = (lane * 2) % _L
    perm_odd = perm_even + 1

    def shuf(x, perm):
        return lax.gather(
            x, perm[:, None],
            lax.GatherDimensionNumbers(
                offset_dims=(), collapsed_slice_dims=(0,), start_index_map=(0,)),
            slice_sizes=(1,),
            mode=lax.GatherScatterMode.PROMISE_IN_BOUNDS)

    def hadd(a, b):
        return jnp.where(lo_half,
                         shuf(a, perm_even) + shuf(a, perm_odd),
                         shuf(b, perm_even) + shuf(b, perm_odd))

    cps = {0: fire(0)}
    for j in range(_NCHUNK):
        if j + 1 < _NCHUNK:
            cps[j + 1] = fire(j + 1)
        for c in cps.pop(j):
            c.wait()
        u_rows = ubufs[j % 2]
        v_rows = vbufs[j % 2]

        def block_body(blk, carry, u_rows=u_rows, v_rows=v_rows, off=j * _CHUNK):
            base_r = blk * _L
            ps = []
            for k in range(_L):
                r = base_r + k
                p = (u_rows[r, pl.ds(0 * _L, _L)] * w0) * v_rows[r, pl.ds(0 * _L, _L)]
                p = p + (u_rows[r, pl.ds(1 * _L, _L)] * w1) * v_rows[r, pl.ds(1 * _L, _L)]
                p = p + (u_rows[r, pl.ds(2 * _L, _L)] * w2) * v_rows[r, pl.ds(2 * _L, _L)]
                p = p + (u_rows[r, pl.ds(3 * _L, _L)] * w3) * v_rows[r, pl.ds(3 * _L, _L)]
                ps.append(p)
            while len(ps) > 1:
                ps = [hadd(ps[i], ps[i + 1]) for i in range(0, len(ps), 2)]
            out_v[pl.ds(off + base_r, _L)] = ps[0]
            return carry

        lax.fori_loop(0, _CHUNK // _L, block_body, 0)

    bv = bias_v[...]
    for i in range(_BPW // _L):
        x = out_v[pl.ds(i * _L, _L)] + bv
        out_v[pl.ds(i * _L, _L)] = 1.0 / (1.0 + jnp.exp(-x))

    pltpu.sync_copy(out_v, out_hbm.at[pl.ds(base, _BPW)])


def kernel(inputs, user_table, item_table, W, b):
    idx = inputs.astype(jnp.int32)
    iu = idx[:, 0].reshape(_NW * _NCHUNK, _CHUNK)
    iv = idx[:, 1].reshape(_NW * _NCHUNK, _CHUNK)
    up = jnp.pad(user_table, ((0, 0), (0, _DP - _D)))
    vp = jnp.pad(item_table, ((0, 0), (0, _DP - _D)))
    w64 = W.reshape(_D).astype(jnp.float32)
    bias = jnp.broadcast_to(b.astype(jnp.float32), (_L,))
    out = _gmf_sc(up, vp, iu, iv, w64, bias)
    return out.reshape(_B, 1)
